# baseline (speedup 1.0000x reference)
import functools

import jax
import jax.numpy as jnp
from jax.experimental import pallas as pl
from jax.experimental.pallas import tpu as pltpu

H = 8
A_DIM = 37
P_DIM = 3 * A_DIM
HALO = 8


def _silu(x):
    return x * jax.nn.sigmoid(x)


def _roll_l(x):
    return pltpu.roll(x, x.shape[1] - 1, 1)


def _roll_r(x):
    return pltpu.roll(x, 1, 1)


def _dot_t(w, x):
    return jax.lax.dot_general(
        w, x, (((0,), (0,)), ((), ())), preferred_element_type=jnp.float32)


def _conv_layer(h, p, cm, wq2, bv, ve, last):
    hn = _roll_l(h)
    pn = _roll_l(p)
    rel = pn - p
    dist = jnp.sqrt(jnp.sum(rel * rel, axis=0, keepdims=True))
    z = (_dot_t(cm[0:H], h) + _dot_t(cm[H:2 * H], hn)
         + _dot_t(cm[2 * H:17], dist) + bv[0:8])
    eh = _silu(z)
    ea = _dot_t(cm[17:25], eh) + bv[8:16]
    ea_m = ea * ve
    nu = ea_m + _roll_r(ea_m)
    nh = _silu(_dot_t(cm[33:41], h) + _dot_t(cm[41:49], nu) + bv[24:32])
    h2 = _dot_t(cm[49:57], nh) + bv[32:40]
    if last:
        return h2, p
    ph = _silu(_dot_t(cm[25:33], ea) + bv[16:24])
    dp = _dot_t(wq2, ph)
    dp_m = dp * ve
    pu = dp_m - _roll_r(dp_m)
    p2 = p + 0.1 * pu
    return h2, p2


def _embed_kernel(ap_ref, am_ref, ec_ref, bv_ref, h0_ref, pos_ref):
    ap = ap_ref[...]
    am = am_ref[...]
    ec = ec_ref[...]
    bv = bv_ref[...]

    ia = jax.lax.broadcasted_iota(jnp.int32, (A_DIM, P_DIM), 0)
    il = jax.lax.broadcasted_iota(jnp.int32, (A_DIM, P_DIM), 1)
    R = (il // 3 == ia).astype(jnp.float32)
    jl = jax.lax.broadcasted_iota(jnp.int32, (3, P_DIM), 0)
    jk = jax.lax.broadcasted_iota(jnp.int32, (3, P_DIM), 1)
    S = (jk % 3 == jl).astype(jnp.float32).T

    mask_rep = _dot_t(R, am)
    wp = ap * mask_rep
    mp = _dot_t(S, wp)
    msum = jnp.sum(am, axis=0, keepdims=True)
    mean_pos = mp / (msum + 1e-8)
    h0 = (_dot_t(ec[0:37], am) + bv[352:360]
          + _dot_t(ec[40:48],
                   _silu(_dot_t(ec[37:40], mean_pos) + bv[360:368]))
          + bv[368:376])
    h0_ref[...] = h0
    pos_ref[...] = mean_pos


def _chain_kernel(hp_ref, hc_ref, hn_ref, pp_ref, pc_ref, pn_ref,
                  cm_ref, wq2_ref, dd_ref, bv_ref, po_ref, mo_ref,
                  *, T, N, G):
    cm = cm_ref[...]
    wq2 = wq2_ref[...]
    dd = dd_ref[...]
    bv = bv_ref[...]
    W = T + 2 * HALO
    t = pl.program_id(0)

    hT = jnp.concatenate(
        [hp_ref[:, T - HALO:], hc_ref[...], hn_ref[:, :HALO]], axis=1)
    posT = jnp.concatenate(
        [pp_ref[:, T - HALO:], pc_ref[...], pn_ref[:, :HALO]], axis=1)

    ids = jax.lax.broadcasted_iota(jnp.int32, (1, W), 1)
    g = ids + (t * T - HALO)
    ve = ((g >= 0) & (g < N - 1)).astype(jnp.float32)

    for i in range(4):
        hT, posT = _conv_layer(hT, posT, cm[57 * i:57 * i + 57], wq2[i],
                               bv[40 * i:40 * i + 40], ve, False)

    zt = _silu(_dot_t(cm[456:464], hT) + bv[320:328])
    zl = _dot_t(cm[464:472], zt) + bv[328:336]
    zf = _silu(_dot_t(cm[472:480], zl) + bv[336:344])
    hT = _dot_t(cm[480:488], zf) + bv[344:352]

    for i in range(4, 8):
        hT, posT = _conv_layer(hT, posT, cm[57 * i:57 * i + 57], wq2[i],
                               bv[40 * i:40 * i + 40], ve, i == 7)

    hF = hT[:, HALO:HALO + T]

    hid = _silu(_dot_t(dd[0:8, :16], hF) + bv[376:392])
    po_ref[...] = _dot_t(dd[8:24], hid) + bv[392:503]
    mo_ref[...] = _dot_t(dd[24:32, :37], hF) + bv[503:540]


def _full_spec(shape):
    nd = len(shape)
    return pl.BlockSpec(shape, lambda t, _n=nd: (0,) * _n)


def kernel(atom_positions, atom_mask, params):
    Bq, Lq, A = atom_mask.shape
    N = Bq * Lq

    apT = atom_positions.reshape(N, P_DIM).T
    amT = atom_mask.reshape(N, A_DIM).T

    We, be = params["node_emb"]
    (Wp1, bp1), (Wp2, bp2) = params["pos_emb"]
    (Wt1, bt1), (Wt2, bt2) = params["to_latent"]
    (Wf1, bf1), (Wf2, bf2) = params["from_latent"]
    (Wd1, bd1), (Wd2, bd2) = params["pos_dec"]
    Wm, bm = params["mask_dec"]

    cm_parts, wq2_parts, bv_parts = [], [], []
    for lp in params["enc"] + params["dec"]:
        (W1e, b1e), (W2e, b2e) = lp["edge"]
        (Wq1, bq1), Wq2 = lp["posm"]
        (Wn1, bn1), (Wn2, bn2) = lp["node"]
        cm_parts += [W1e, W2e, Wq1, Wn1, Wn2]
        wq2_parts.append(Wq2)
        bv_parts += [b1e, b2e, bq1, bn1, bn2]
    cm_parts += [Wt1, Wt2, Wf1, Wf2]
    bv_parts += [bt1, bt2, bf1, bf2, be, bp1, bp2, bd1, bd2, bm]

    CM = jnp.concatenate(cm_parts, axis=0)
    WQ2 = jnp.stack(wq2_parts)
    DD = jnp.concatenate(
        [jnp.pad(Wd1, ((0, 0), (0, P_DIM - 16))), Wd2,
         jnp.pad(Wm, ((0, 0), (0, P_DIM - A_DIM)))], axis=0)
    BV = jnp.concatenate(bv_parts)[:, None]
    EC = jnp.concatenate([We, Wp1, Wp2], axis=0)

    T1 = 8192 if N % 8192 == 0 else N
    G1 = N // T1
    h0T, posT = pl.pallas_call(
        _embed_kernel,
        grid=(G1,),
        in_specs=[pl.BlockSpec((P_DIM, T1), lambda t: (0, t)),
                  pl.BlockSpec((A_DIM, T1), lambda t: (0, t)),
                  _full_spec(EC.shape), _full_spec(BV.shape)],
        out_specs=[pl.BlockSpec((H, T1), lambda t: (0, t)),
                   pl.BlockSpec((3, T1), lambda t: (0, t))],
        out_shape=[jax.ShapeDtypeStruct((H, N), jnp.float32),
                   jax.ShapeDtypeStruct((3, N), jnp.float32)],
    )(apT, amT, EC, BV)

    T2 = 8192 if N % 8192 == 0 else N
    G2 = N // T2

    def prv(t):
        return (0, jnp.maximum(t - 1, 0))

    def cur(t):
        return (0, t)

    def nxt(t):
        return (0, jnp.minimum(t + 1, G2 - 1))

    po, mo = pl.pallas_call(
        functools.partial(_chain_kernel, T=T2, N=N, G=G2),
        grid=(G2,),
        in_specs=[pl.BlockSpec((H, T2), prv),
                  pl.BlockSpec((H, T2), cur),
                  pl.BlockSpec((H, T2), nxt),
                  pl.BlockSpec((3, T2), prv),
                  pl.BlockSpec((3, T2), cur),
                  pl.BlockSpec((3, T2), nxt),
                  _full_spec(CM.shape), _full_spec(WQ2.shape),
                  _full_spec(DD.shape), _full_spec(BV.shape)],
        out_specs=[pl.BlockSpec((P_DIM, T2), lambda t: (0, t)),
                   pl.BlockSpec((A_DIM, T2), lambda t: (0, t))],
        out_shape=[jax.ShapeDtypeStruct((P_DIM, N), jnp.float32),
                   jax.ShapeDtypeStruct((A_DIM, N), jnp.float32)],
    )(h0T, h0T, h0T, posT, posT, posT, CM, WQ2, DD, BV)

    return (po.T.reshape(Bq, Lq, A, 3), mo.T.reshape(Bq, Lq, A))
